# baseline (speedup 1.0000x reference)
import functools

import jax
import jax.numpy as jnp
from jax import lax
from jax.experimental import pallas as pl
from jax.experimental.pallas import tpu as pltpu, tpu_sc as plsc

N = 10000
NPAD = 10240
E = 320000
NC = 2
NS = 16
NW = NC * NS
EPW = E // NW
K = 80
NCHUNK = EPW // K
RPW = NPAD // NS
RBLK = K
NRB = RPW // RBLK
_PHASES = ((0, 0, 63), (63, 62, 62))
_SRCWIN = 63

_sc_params = pltpu.CompilerParams(use_tc_tiling_on_sc=False)

_mesh = plsc.VectorSubcoreMesh(core_axis_name="c", subcore_axis_name="s",
                               num_cores=NC, num_subcores=NS)


def _zero_block(ref, nrows, ncol16):
    zero16 = jnp.zeros((16,), jnp.float32)

    def body(i, _):
        for q in range(ncol16):
            ref[i, pl.ds(q * 16, 16)] = zero16
        return 0

    lax.fori_loop(0, nrows, body, 0)


@functools.partial(
    pl.kernel,
    out_type=jax.ShapeDtypeStruct((NC, NPAD, 16), jnp.float32),
    mesh=_mesh,
    compiler_params=_sc_params,
    scratch_types=[
        pltpu.VMEM((NCHUNK, K), jnp.int32),
        pltpu.VMEM((NCHUNK, K), jnp.float32),
        pltpu.VMEM((K, 16), jnp.float32),
        pltpu.VMEM((RPW, 16), jnp.float32),
        pltpu.VMEM_SHARED((NPAD, 16), jnp.float32),
    ],
)
def _deg_kernel(dst_hbm, ew_hbm, out_hbm, dst_v, ew_v, buf, zblk, hist):
    c = lax.axis_index("c")
    s = lax.axis_index("s")
    chunk = c * NS + s
    pltpu.sync_copy(dst_hbm.at[chunk], dst_v)
    pltpu.sync_copy(ew_hbm.at[chunk], ew_v)

    _zero_block(zblk, RPW, 1)
    pltpu.sync_copy(zblk, hist.at[pl.ds(s * RPW, RPW)])

    ones_f = jnp.ones((16,), jnp.float32)
    lane_lo = lax.broadcasted_iota(jnp.int32, (16,), 0) < 8
    plsc.subcore_barrier()

    def chunk_body(j, _):
        def grp_body(g, _):
            wv = ew_v[j, pl.ds(g * 16, 16)]
            for l in range(16):
                buf[g * 16 + l, :] = jnp.where(
                    lane_lo, jnp.broadcast_to(wv[l], (16,)), ones_f)
            return 0
        lax.fori_loop(0, K // 16, grp_body, 0)
        pltpu.sync_copy(buf, hist.at[dst_v.at[j]], add=True)
        return 0

    lax.fori_loop(0, NCHUNK, chunk_body, 0)
    plsc.subcore_barrier()

    pltpu.sync_copy(hist.at[pl.ds(s * RPW, RPW)],
                    out_hbm.at[c, pl.ds(s * RPW, RPW)])


def _make_agg_kernel(weighted):
    scratch = (
        [pltpu.VMEM((_SRCWIN, K), jnp.int32)]
        + [pltpu.VMEM((K, 128), jnp.float32)] * 4
        + [pltpu.VMEM((1, K), jnp.int32)] * 4
        + [pltpu.VMEM((1, K), jnp.float32)] * 4
        + [pltpu.VMEM_SHARED((NPAD, 128), jnp.float32)]
        + [pltpu.SemaphoreType.DMA] * 12
    )

    def body(table_hbm, src_hbm, dst_hbm, ew_hbm, out_hbm,
             src_v, r0, r1, r2, r3, d0, d1, d2, d3, w0, w1, w2, w3, acc,
             sr0, sr1, sr2, sr3, sd0, sd1, sd2, sd3, sw0, sw1, sw2, sw3):
        c = lax.axis_index("c")
        s = lax.axis_index("s")
        chunk = c * NS + s

        rows = [r0, r1, r2, r3]
        dsts = [d0, d1, d2, d3]
        ews = [w0, w1, w2, w3]
        rsem = [sr0, sr1, sr2, sr3]
        dsem = [sd0, sd1, sd2, sd3]
        wsem = [sw0, sw1, sw2, sw3]

        _zero_block(r0, RBLK, 8)
        for t in range(NRB):
            pltpu.sync_copy(r0, acc.at[pl.ds(s * RPW + t * RBLK, RBLK)])
        plsc.subcore_barrier()

        def fetch(base, o, j, b):
            pltpu.async_copy(table_hbm.at[src_v.at[o + j]], rows[b], rsem[b])
            pltpu.async_copy(dst_hbm.at[chunk].at[pl.ds(base + j, 1)],
                             dsts[b], dsem[b])
            if weighted:
                pltpu.async_copy(ew_hbm.at[chunk].at[pl.ds(base + j, 1)],
                                 ews[b], wsem[b])

        def wait_proc(b):
            pltpu.make_async_copy(table_hbm.at[src_v.at[0]], rows[b],
                                  rsem[b]).wait()
            pltpu.make_async_copy(dst_hbm.at[0].at[pl.ds(0, 1)], dsts[b],
                                  dsem[b]).wait()
            if weighted:
                pltpu.make_async_copy(ew_hbm.at[0].at[pl.ds(0, 1)], ews[b],
                                      wsem[b]).wait()

                def grp_body(g, _):
                    wv = ews[b][0, pl.ds(g * 16, 16)]
                    for l in range(16):
                        w = wv[l]
                        r = g * 16 + l
                        for q in range(8):
                            sl = pl.ds(q * 16, 16)
                            rows[b][r, sl] = rows[b][r, sl] * w
                    return 0

                lax.fori_loop(0, K // 16, grp_body, 0)
            pltpu.sync_copy(rows[b], acc.at[dsts[b].at[0]], add=True)

        for pidx, (base, w, C) in enumerate(_PHASES):
            o = base - w
            pltpu.sync_copy(src_hbm.at[chunk * len(_PHASES) + pidx], src_v)
            for p in range(3):
                fetch(base, o, p, p)

            def quad_body(i, _, base=base, o=o):
                j = 4 * i
                fetch(base, o, j + 3, 3)
                wait_proc(0)
                fetch(base, o, j + 4, 0)
                wait_proc(1)
                fetch(base, o, j + 5, 1)
                wait_proc(2)
                fetch(base, o, j + 6, 2)
                wait_proc(3)
                return 0

            nfull = (C - 3) // 4
            lax.fori_loop(0, nfull, quad_body, 0)
            for j in range(4 * nfull, C):
                if j + 3 < C:
                    fetch(base, o, j + 3, (j + 3) % 4)
                wait_proc(j % 4)
        plsc.subcore_barrier()

        pltpu.sync_copy(acc.at[pl.ds(s * RPW, RPW)],
                        out_hbm.at[c, pl.ds(s * RPW, RPW)])

    return pl.kernel(
        body,
        out_type=jax.ShapeDtypeStruct((NC, NPAD, 128), jnp.float32),
        mesh=_mesh,
        compiler_params=_sc_params,
        scratch_types=scratch,
    )


_agg_w = _make_agg_kernel(True)
_agg_1 = _make_agg_kernel(False)


_BR = 1000
_NB = N // _BR


def _scales_body(hist_ref, x_ref, xs_ref, aux_ref):
    h = hist_ref[...]
    degw = h[0, :, 0] + h[1, :, 0] + 1.0
    deg1 = h[0, :, 8] + h[1, :, 8] + 1.0
    disw = jnp.where(degw > 0, lax.rsqrt(degw), 0.0)
    dis1 = jnp.where(deg1 > 0, lax.rsqrt(deg1), 0.0)
    aux_ref[...] = jnp.stack([disw, disw * disw, dis1, dis1 * dis1], axis=1)
    xs_ref[...] = x_ref[...] * disw[:, None]


def _scales_call(hist, x):
    return pl.pallas_call(
        _scales_body,
        grid=(_NB,),
        in_specs=[
            pl.BlockSpec((NC, _BR, 16), lambda i: (0, i, 0)),
            pl.BlockSpec((_BR, 128), lambda i: (i, 0)),
        ],
        out_specs=[
            pl.BlockSpec((_BR, 128), lambda i: (i, 0)),
            pl.BlockSpec((_BR, 4), lambda i: (i, 0)),
        ],
        out_shape=[
            jax.ShapeDtypeStruct((N, 128), jnp.float32),
            jax.ShapeDtypeStruct((N, 4), jnp.float32),
        ],
    )(hist, x)


def _make_conv_body(col, relu, scale_col):
    def body(p_ref, xin_ref, aux_ref, w_ref, b_ref, out_ref, *scaled_ref):
        aux = aux_ref[...]
        t = ((p_ref[0] + p_ref[1]) * aux[:, col][:, None]
             + xin_ref[...] * aux[:, col + 1][:, None])
        o = jnp.dot(t, w_ref[...], preferred_element_type=jnp.float32) + b_ref[0, :]
        if relu:
            o = jnp.maximum(o, 0.0)
        out_ref[...] = o
        if scale_col is not None:
            scaled_ref[0][...] = o * aux[:, scale_col][:, None]

    return body


def _conv_call(p, xin, aux, w, b, col, relu, scale_col):
    kout = w.shape[1]
    out_shape = [jax.ShapeDtypeStruct((N, kout), jnp.float32)]
    out_specs = [pl.BlockSpec((_BR, kout), lambda i: (i, 0))]
    if scale_col is not None:
        out_shape.append(jax.ShapeDtypeStruct((N, kout), jnp.float32))
        out_specs.append(pl.BlockSpec((_BR, kout), lambda i: (i, 0)))
    res = pl.pallas_call(
        _make_conv_body(col, relu, scale_col),
        grid=(_NB,),
        in_specs=[
            pl.BlockSpec((NC, _BR, 128), lambda i: (0, i, 0)),
            pl.BlockSpec((_BR, 128), lambda i: (i, 0)),
            pl.BlockSpec((_BR, 4), lambda i: (i, 0)),
            pl.BlockSpec((128, kout), lambda i: (0, 0)),
            pl.BlockSpec((1, kout), lambda i: (0, 0)),
        ],
        out_specs=out_specs,
        out_shape=out_shape,
    )(p, xin, aux, w, b.reshape(1, kout))
    return res if scale_col is not None else (res[0], None)


def _heads_body(p_ref, xin_ref, aux_ref, wmu_ref, bmu_ref, wls_ref, bls_ref,
                mu_ref, ls_ref):
    aux = aux_ref[...]
    agg = ((p_ref[0] + p_ref[1]) * aux[:, 2][:, None]
           + xin_ref[...] * aux[:, 3][:, None])
    mu_ref[...] = jnp.dot(agg, wmu_ref[...],
                          preferred_element_type=jnp.float32) + bmu_ref[0, :]
    ls_ref[...] = jnp.dot(agg, wls_ref[...],
                          preferred_element_type=jnp.float32) + bls_ref[0, :]


def _heads_call(p, xin, aux, wmu, bmu, wls, bls):
    kout = wmu.shape[1]
    return pl.pallas_call(
        _heads_body,
        grid=(_NB,),
        in_specs=[
            pl.BlockSpec((NC, _BR, 128), lambda i: (0, i, 0)),
            pl.BlockSpec((_BR, 128), lambda i: (i, 0)),
            pl.BlockSpec((_BR, 4), lambda i: (i, 0)),
            pl.BlockSpec((128, kout), lambda i: (0, 0)),
            pl.BlockSpec((1, kout), lambda i: (0, 0)),
            pl.BlockSpec((128, kout), lambda i: (0, 0)),
            pl.BlockSpec((1, kout), lambda i: (0, 0)),
        ],
        out_specs=[
            pl.BlockSpec((_BR, kout), lambda i: (i, 0)),
            pl.BlockSpec((_BR, kout), lambda i: (i, 0)),
        ],
        out_shape=[
            jax.ShapeDtypeStruct((N, kout), jnp.float32),
            jax.ShapeDtypeStruct((N, kout), jnp.float32),
        ],
    )(p, xin, aux, wmu, bmu.reshape(1, kout), wls, bls.reshape(1, kout))


def kernel(x, edge_index, edge_weight, W1, b1, W2, b2, W_mu, b_mu, W_ls, b_ls):
    ei = edge_index.astype(jnp.int32)
    src3 = ei[0].reshape(NW, NCHUNK, K)
    dst3 = ei[1].reshape(NW, NCHUNK, K)
    ew3 = edge_weight.reshape(NW, NCHUNK, K)
    srcw = jnp.stack(
        [src3[:, w:w + _SRCWIN] for _, w, _ in _PHASES], axis=1,
    ).reshape(NW * len(_PHASES), _SRCWIN, K)

    hist = _deg_kernel(dst3, ew3)
    xs0, aux = _scales_call(hist, x)

    p1 = _agg_w(xs0, srcw, dst3, ew3)
    h1, h1s = _conv_call(p1, x, aux, W1, b1, col=0, relu=True, scale_col=0)

    p2 = _agg_w(h1s, srcw, dst3, ew3)
    h2, h2s = _conv_call(p2, h1, aux, W2, b2, col=0, relu=False, scale_col=2)

    p3 = _agg_1(h2s, srcw, dst3, ew3)
    mu, ls = _heads_call(p3, h2, aux, W_mu, b_mu, W_ls, b_ls)
    return (mu, ls)


# speedup vs baseline: 1.0379x; 1.0379x over previous
import functools

import jax
import jax.numpy as jnp
from jax import lax
from jax.experimental import pallas as pl
from jax.experimental.pallas import tpu as pltpu, tpu_sc as plsc

N = 10000
NPAD = 10240
E = 320000
NC = 2
NS = 16
NW = NC * NS
EPW = E // NW
K = 80
NCHUNK = EPW // K
RPW = NPAD // NS
RBLK = K
NRB = RPW // RBLK

_sc_params = pltpu.CompilerParams(use_tc_tiling_on_sc=False)

_mesh = plsc.VectorSubcoreMesh(core_axis_name="c", subcore_axis_name="s",
                               num_cores=NC, num_subcores=NS)


def _zero_block(ref, nrows, ncol16):
    zero16 = jnp.zeros((16,), jnp.float32)

    def body(i, _):
        for q in range(ncol16):
            ref[i, pl.ds(q * 16, 16)] = zero16
        return 0

    lax.fori_loop(0, nrows, body, 0)


@functools.partial(
    pl.kernel,
    out_type=jax.ShapeDtypeStruct((NC, NPAD, 16), jnp.float32),
    mesh=_mesh,
    compiler_params=_sc_params,
    scratch_types=[
        pltpu.VMEM((NCHUNK, K), jnp.int32),
        pltpu.VMEM((NCHUNK, K), jnp.float32),
        pltpu.VMEM((K, 16), jnp.float32),
        pltpu.VMEM((RPW, 16), jnp.float32),
        pltpu.VMEM_SHARED((NPAD, 16), jnp.float32),
    ],
)
def _deg_kernel(dst_hbm, ew_hbm, out_hbm, dst_v, ew_v, buf, zblk, hist):
    c = lax.axis_index("c")
    s = lax.axis_index("s")
    chunk = c * NS + s
    pltpu.sync_copy(dst_hbm.at[chunk], dst_v)
    pltpu.sync_copy(ew_hbm.at[chunk], ew_v)

    _zero_block(zblk, RPW, 1)
    pltpu.sync_copy(zblk, hist.at[pl.ds(s * RPW, RPW)])

    ones_f = jnp.ones((16,), jnp.float32)
    lane_lo = lax.broadcasted_iota(jnp.int32, (16,), 0) < 8
    plsc.subcore_barrier()

    def chunk_body(j, _):
        def grp_body(g, _):
            wv = ew_v[j, pl.ds(g * 16, 16)]
            for l in range(16):
                buf[g * 16 + l, :] = jnp.where(
                    lane_lo, jnp.broadcast_to(wv[l], (16,)), ones_f)
            return 0
        lax.fori_loop(0, K // 16, grp_body, 0)
        pltpu.sync_copy(buf, hist.at[dst_v.at[j]], add=True)
        return 0

    lax.fori_loop(0, NCHUNK, chunk_body, 0)
    plsc.subcore_barrier()

    pltpu.sync_copy(hist.at[pl.ds(s * RPW, RPW)],
                    out_hbm.at[c, pl.ds(s * RPW, RPW)])


def _make_agg_kernel(weighted):
    scratch = (
        [pltpu.VMEM((NCHUNK, K), jnp.int32)]
        + [pltpu.VMEM((K, 128), jnp.float32)] * 3
        + [pltpu.VMEM((1, K), jnp.int32)] * 3
        + [pltpu.VMEM((1, K), jnp.float32)] * 3
        + [pltpu.VMEM_SHARED((NPAD, 128), jnp.float32)]
        + [pltpu.SemaphoreType.DMA] * 9
    )

    def body(table_hbm, src_hbm, dst_hbm, ew_hbm, out_hbm,
             src_v, r0, r1, r2, d0, d1, d2, w0, w1, w2, acc,
             sr0, sr1, sr2, sd0, sd1, sd2, sw0, sw1, sw2):
        c = lax.axis_index("c")
        s = lax.axis_index("s")
        chunk = c * NS + s
        pltpu.sync_copy(src_hbm.at[chunk], src_v)

        rows = [r0, r1, r2]
        dsts = [d0, d1, d2]
        ews = [w0, w1, w2]
        rsem = [sr0, sr1, sr2]
        dsem = [sd0, sd1, sd2]
        wsem = [sw0, sw1, sw2]

        _zero_block(r0, RBLK, 8)
        for t in range(NRB):
            pltpu.sync_copy(r0, acc.at[pl.ds(s * RPW + t * RBLK, RBLK)])
        plsc.subcore_barrier()

        def fetch(j, b):
            pltpu.async_copy(table_hbm.at[src_v.at[j]], rows[b], rsem[b])
            pltpu.async_copy(dst_hbm.at[chunk].at[pl.ds(j, 1)], dsts[b],
                             dsem[b])
            if weighted:
                pltpu.async_copy(ew_hbm.at[chunk].at[pl.ds(j, 1)], ews[b],
                                 wsem[b])

        def wait_proc(b):
            pltpu.make_async_copy(table_hbm.at[src_v.at[0]], rows[b],
                                  rsem[b]).wait()
            pltpu.make_async_copy(dst_hbm.at[0].at[pl.ds(0, 1)], dsts[b],
                                  dsem[b]).wait()
            if weighted:
                pltpu.make_async_copy(ew_hbm.at[0].at[pl.ds(0, 1)], ews[b],
                                      wsem[b]).wait()

                def grp_body(g, _):
                    wv = ews[b][0, pl.ds(g * 16, 16)]
                    for l in range(16):
                        w = wv[l]
                        r = g * 16 + l
                        for q in range(8):
                            sl = pl.ds(q * 16, 16)
                            rows[b][r, sl] = rows[b][r, sl] * w
                    return 0

                lax.fori_loop(0, K // 16, grp_body, 0)
            pltpu.sync_copy(rows[b], acc.at[dsts[b].at[0]], add=True)

        fetch(0, 0)
        fetch(1, 1)

        def tri_body(i, _):
            j = 3 * i
            fetch(j + 2, 2)
            wait_proc(0)
            fetch(j + 3, 0)
            wait_proc(1)
            fetch(j + 4, 1)
            wait_proc(2)
            return 0

        lax.fori_loop(0, (NCHUNK - 2) // 3, tri_body, 0)
        wait_proc(0)
        wait_proc(1)
        plsc.subcore_barrier()

        pltpu.sync_copy(acc.at[pl.ds(s * RPW, RPW)],
                        out_hbm.at[c, pl.ds(s * RPW, RPW)])

    return pl.kernel(
        body,
        out_type=jax.ShapeDtypeStruct((NC, NPAD, 128), jnp.float32),
        mesh=_mesh,
        compiler_params=_sc_params,
        scratch_types=scratch,
    )


_agg_w = _make_agg_kernel(True)
_agg_1 = _make_agg_kernel(False)


_BR = 1000
_NB = N // _BR


def _scales_body(hist_ref, x_ref, xs_ref, aux_ref):
    h = hist_ref[...]
    degw = h[0, :, 0] + h[1, :, 0] + 1.0
    deg1 = h[0, :, 8] + h[1, :, 8] + 1.0
    disw = jnp.where(degw > 0, lax.rsqrt(degw), 0.0)
    dis1 = jnp.where(deg1 > 0, lax.rsqrt(deg1), 0.0)
    aux_ref[...] = jnp.stack([disw, disw * disw, dis1, dis1 * dis1], axis=1)
    xs_ref[...] = x_ref[...] * disw[:, None]


def _scales_call(hist, x):
    return pl.pallas_call(
        _scales_body,
        grid=(_NB,),
        in_specs=[
            pl.BlockSpec((NC, _BR, 16), lambda i: (0, i, 0)),
            pl.BlockSpec((_BR, 128), lambda i: (i, 0)),
        ],
        out_specs=[
            pl.BlockSpec((_BR, 128), lambda i: (i, 0)),
            pl.BlockSpec((_BR, 4), lambda i: (i, 0)),
        ],
        out_shape=[
            jax.ShapeDtypeStruct((N, 128), jnp.float32),
            jax.ShapeDtypeStruct((N, 4), jnp.float32),
        ],
    )(hist, x)


def _make_conv_body(col, relu, scale_col):
    def body(p_ref, xin_ref, aux_ref, w_ref, b_ref, out_ref, *scaled_ref):
        aux = aux_ref[...]
        t = ((p_ref[0] + p_ref[1]) * aux[:, col][:, None]
             + xin_ref[...] * aux[:, col + 1][:, None])
        o = jnp.dot(t, w_ref[...], preferred_element_type=jnp.float32) + b_ref[0, :]
        if relu:
            o = jnp.maximum(o, 0.0)
        out_ref[...] = o
        if scale_col is not None:
            scaled_ref[0][...] = o * aux[:, scale_col][:, None]

    return body


def _conv_call(p, xin, aux, w, b, col, relu, scale_col):
    kout = w.shape[1]
    out_shape = [jax.ShapeDtypeStruct((N, kout), jnp.float32)]
    out_specs = [pl.BlockSpec((_BR, kout), lambda i: (i, 0))]
    if scale_col is not None:
        out_shape.append(jax.ShapeDtypeStruct((N, kout), jnp.float32))
        out_specs.append(pl.BlockSpec((_BR, kout), lambda i: (i, 0)))
    res = pl.pallas_call(
        _make_conv_body(col, relu, scale_col),
        grid=(_NB,),
        in_specs=[
            pl.BlockSpec((NC, _BR, 128), lambda i: (0, i, 0)),
            pl.BlockSpec((_BR, 128), lambda i: (i, 0)),
            pl.BlockSpec((_BR, 4), lambda i: (i, 0)),
            pl.BlockSpec((128, kout), lambda i: (0, 0)),
            pl.BlockSpec((1, kout), lambda i: (0, 0)),
        ],
        out_specs=out_specs,
        out_shape=out_shape,
    )(p, xin, aux, w, b.reshape(1, kout))
    return res if scale_col is not None else (res[0], None)


def _heads_body(p_ref, xin_ref, aux_ref, wmu_ref, bmu_ref, wls_ref, bls_ref,
                mu_ref, ls_ref):
    aux = aux_ref[...]
    agg = ((p_ref[0] + p_ref[1]) * aux[:, 2][:, None]
           + xin_ref[...] * aux[:, 3][:, None])
    mu_ref[...] = jnp.dot(agg, wmu_ref[...],
                          preferred_element_type=jnp.float32) + bmu_ref[0, :]
    ls_ref[...] = jnp.dot(agg, wls_ref[...],
                          preferred_element_type=jnp.float32) + bls_ref[0, :]


def _heads_call(p, xin, aux, wmu, bmu, wls, bls):
    kout = wmu.shape[1]
    return pl.pallas_call(
        _heads_body,
        grid=(_NB,),
        in_specs=[
            pl.BlockSpec((NC, _BR, 128), lambda i: (0, i, 0)),
            pl.BlockSpec((_BR, 128), lambda i: (i, 0)),
            pl.BlockSpec((_BR, 4), lambda i: (i, 0)),
            pl.BlockSpec((128, kout), lambda i: (0, 0)),
            pl.BlockSpec((1, kout), lambda i: (0, 0)),
            pl.BlockSpec((128, kout), lambda i: (0, 0)),
            pl.BlockSpec((1, kout), lambda i: (0, 0)),
        ],
        out_specs=[
            pl.BlockSpec((_BR, kout), lambda i: (i, 0)),
            pl.BlockSpec((_BR, kout), lambda i: (i, 0)),
        ],
        out_shape=[
            jax.ShapeDtypeStruct((N, kout), jnp.float32),
            jax.ShapeDtypeStruct((N, kout), jnp.float32),
        ],
    )(p, xin, aux, wmu, bmu.reshape(1, kout), wls, bls.reshape(1, kout))


def kernel(x, edge_index, edge_weight, W1, b1, W2, b2, W_mu, b_mu, W_ls, b_ls):
    ei = edge_index.astype(jnp.int32)
    src3 = ei[0].reshape(NW, NCHUNK, K)
    dst3 = ei[1].reshape(NW, NCHUNK, K)
    ew3 = edge_weight.reshape(NW, NCHUNK, K)

    hist = _deg_kernel(dst3, ew3)
    xs0, aux = _scales_call(hist, x)

    p1 = _agg_w(xs0, src3, dst3, ew3)
    h1, h1s = _conv_call(p1, x, aux, W1, b1, col=0, relu=True, scale_col=0)

    p2 = _agg_w(h1s, src3, dst3, ew3)
    h2, h2s = _conv_call(p2, h1, aux, W2, b2, col=0, relu=False, scale_col=2)

    p3 = _agg_1(h2s, src3, dst3, ew3)
    mu, ls = _heads_call(p3, h2, aux, W_mu, b_mu, W_ls, b_ls)
    return (mu, ls)
